# Initial kernel scaffold; baseline (speedup 1.0000x reference)
import jax
import jax.numpy as jnp
from jax.experimental import pallas as pl


def kernel(x, edge_index, edge_values, Wk, Wq, Wv, gamma, beta):
    raise NotImplementedError("write your pallas kernel here")


# speedup vs baseline: 3.2251x; 3.2251x over previous
import functools

import jax
import jax.numpy as jnp
from jax import lax
from jax.experimental import pallas as pl
from jax.experimental.pallas import tpu as pltpu
from jax.experimental.pallas import tpu_sc as plsc

L, N, E, D, KD = 2, 10000, 320000, 128, 16
DECAY = 0.5

NC, NS = 2, 16
BK = 128
EPT = -(-E // (NS * BK)) * BK
E2 = EPT * NS
NBATCH = EPT // BK
CR = 80
NCHUNK = N // CR


def _sc_spmm2(x_flat, src, dst, ev):
    mesh = plsc.VectorSubcoreMesh(core_axis_name="c", subcore_axis_name="s")

    @functools.partial(
        pl.kernel,
        out_type=[
            jax.ShapeDtypeStruct((L * N, D), jnp.float32),
            jax.ShapeDtypeStruct((L * N, D), jnp.float32),
        ],
        mesh=mesh,
        scratch_types=[
            pltpu.VMEM_SHARED((N, D), jnp.float32),
            pltpu.VMEM((BK,), jnp.int32),
            pltpu.VMEM((BK,), jnp.int32),
            pltpu.VMEM((BK,), jnp.float32),
            pltpu.VMEM((BK, D), jnp.float32),
            pltpu.SemaphoreType.DMA,
        ],
    )
    def k(x_hbm, src_hbm, dst_hbm, ev_hbm, y1_hbm, y2_hbm,
          accum, src_v, dst_v, ev_v, rows_v, sem):
        l = lax.axis_index("c")
        s = lax.axis_index("s")
        zeros16 = jnp.zeros((16,), jnp.float32)
        clo = (NCHUNK * s) // NS
        chi = (NCHUNK * (s + 1)) // NS

        def run_iter(tab_hbm, out_hbm):
            def zfill(j, _):
                for t in range(D // 16):
                    rows_v[j, pl.ds(16 * t, 16)] = zeros16
                return 0
            lax.fori_loop(0, CR, zfill, 0)

            def clear(c, _):
                r = pl.multiple_of(c * CR, 8)
                pltpu.sync_copy(rows_v.at[pl.ds(0, CR)],
                                accum.at[pl.ds(r, CR)])
                return 0
            lax.fori_loop(clo, chi, clear, 0)
            plsc.subcore_barrier()

            def batch(b, _):
                base = s * EPT + b * BK
                pltpu.sync_copy(src_hbm.at[pl.ds(base, BK)], src_v)
                pltpu.sync_copy(dst_hbm.at[pl.ds(base, BK)], dst_v)
                pltpu.sync_copy(ev_hbm.at[pl.ds(base, BK)], ev_v)
                off = (l * N).astype(jnp.int32)
                for t in range(BK // 16):
                    sl = pl.ds(16 * t, 16)
                    src_v[sl] = src_v[sl] + off
                pltpu.async_copy(tab_hbm.at[src_v], rows_v, sem).wait()

                def scale_grp(g, _):
                    evg = ev_v[pl.ds(16 * g, 16)] * DECAY
                    for lane in range(16):
                        sc = evg[lane]
                        i = 16 * g + lane
                        for t in range(D // 16):
                            sl = pl.ds(16 * t, 16)
                            rows_v[i, sl] = rows_v[i, sl] * sc
                    return 0
                lax.fori_loop(0, BK // 16, scale_grp, 0)
                pltpu.sync_copy(rows_v, accum.at[dst_v], add=True)
                return 0
            lax.fori_loop(0, NBATCH, batch, 0)
            plsc.subcore_barrier()

            def copyout(c, _):
                r = pl.multiple_of(c * CR, 8)
                pltpu.sync_copy(accum.at[pl.ds(r, CR)],
                                out_hbm.at[pl.ds(l * N + r, CR)])
                return 0
            lax.fori_loop(clo, chi, copyout, 0)
            plsc.subcore_barrier()

        run_iter(x_hbm, y1_hbm)
        run_iter(y1_hbm, y2_hbm)

    return k(x_flat, src, dst, ev)


def _tc_dense(x_flat, y1, y2, Wk, Wq, Wv, gamma, beta):
    BN = 1000

    def body(x_ref, y1_ref, y2_ref, wk_ref, wq_ref, wv_ref, g_ref, b_ref,
             o_ref):
        xb = x_ref[...]
        y1b = y1_ref[...]
        y2b = y2_ref[...]
        wk = wk_ref[...]
        wq = wq_ref[...]
        dn = (((1,), (1,)), ((), ()))

        def wpart(v):
            kp = lax.dot_general(v, wk, dn, preferred_element_type=jnp.float32)
            qp = lax.dot_general(v, wq, dn, preferred_element_type=jnp.float32)
            return jnp.sum(kp * qp, axis=1, keepdims=True) * (1.0 / KD)

        w = wpart(xb) + wpart(y1b) + wpart(y2b)
        xo = xb + y1b + y2b
        vals = lax.dot_general(xo, wv_ref[...], dn,
                               preferred_element_type=jnp.float32)
        vw = vals * w
        mu = jnp.mean(vw, axis=1, keepdims=True)
        dv = vw - mu
        var = jnp.mean(dv * dv, axis=1, keepdims=True)
        o_ref[...] = dv * lax.rsqrt(var + 1e-5) * g_ref[...] + b_ref[...]

    return pl.pallas_call(
        body,
        grid=(L * N // BN,),
        in_specs=[
            pl.BlockSpec((BN, D), lambda i: (i, 0)),
            pl.BlockSpec((BN, D), lambda i: (i, 0)),
            pl.BlockSpec((BN, D), lambda i: (i, 0)),
            pl.BlockSpec((KD, D), lambda i: (0, 0)),
            pl.BlockSpec((KD, D), lambda i: (0, 0)),
            pl.BlockSpec((D, D), lambda i: (0, 0)),
            pl.BlockSpec((1, D), lambda i: (0, 0)),
            pl.BlockSpec((1, D), lambda i: (0, 0)),
        ],
        out_specs=pl.BlockSpec((BN, D), lambda i: (i, 0)),
        out_shape=jax.ShapeDtypeStruct((L * N, D), jnp.float32),
    )(x_flat, y1, y2, Wk, Wq, Wv, gamma, beta)


def kernel(x, edge_index, edge_values, Wk, Wq, Wv, gamma, beta):
    x_flat = x.reshape(L * N, D)
    pad = E2 - E
    src = jnp.concatenate([edge_index[1], jnp.zeros((pad,), jnp.int32)])
    dst = jnp.concatenate([edge_index[0], jnp.zeros((pad,), jnp.int32)])
    ev = jnp.concatenate([edge_values, jnp.zeros((pad,), jnp.float32)])
    y1, y2 = _sc_spmm2(x_flat, src, dst, ev)
    out = _tc_dense(x_flat, y1, y2, Wk, Wq, Wv,
                    gamma.reshape(1, D), beta.reshape(1, D))
    return out.reshape(L, N, D)
